# Initial kernel scaffold; baseline (speedup 1.0000x reference)
import jax
import jax.numpy as jnp
from jax.experimental import pallas as pl


def kernel(author_ids, topic_ids, auth_cnts, topic_cnts, paper_ids, id_maps, edge_indexs, paper_sets, author_sets, topic_sets, topic_embs, au_embs, paper_embs, topic_W, topic_b, au_W, au_b, paper_W, paper_b, conv_W, conv_b, p1_W, p1_b, p2_W, p2_b, p3_W, p3_b):
    raise NotImplementedError("write your pallas kernel here")


# speedup vs baseline: 45.6437x; 45.6437x over previous
import functools
import numpy as np
import jax
import jax.numpy as jnp
from jax import lax
from jax.experimental import pallas as pl
from jax.experimental.pallas import tpu as pltpu
from jax.experimental.pallas import tpu_sc as plsc

B = 4
N = 10000
E = 320000
D = 128
NS = 16
NC = 2
NCHUNK = 157
SETP = 10112
NSARR = 10240
ZROW = 9999
TROWS = 10000
MCAP = 158 * 128


def _rsqrt16(d):
    i = plsc.bitcast(d, jnp.int32)
    i = jnp.full((16,), 0x5F3759DF, jnp.int32) - lax.shift_right_arithmetic(i, 1)
    y = plsc.bitcast(i, jnp.float32)
    for _ in range(4):
        y = y * (np.float32(1.5) - np.float32(0.5) * d * y * y)
    return y


def _tbl_body(x_ref, w_ref, b_ref, cw_ref, o_ref):
    x = x_ref[0]
    h = jnp.tanh(jnp.dot(x, w_ref[...], preferred_element_type=jnp.float32)
                 + b_ref[...])
    o_ref[0] = jnp.dot(h, cw_ref[...], preferred_element_type=jnp.float32)


def _make_table(embs, W, bvec, conv_W, rows_pad):
    Bn, R, K = embs.shape
    Kp = max(128, ((K + 127) // 128) * 128)
    x = jnp.pad(embs, ((0, 0), (0, rows_pad - R), (0, Kp - K)))
    Wp = jnp.pad(W, ((0, Kp - K), (0, 0)))
    call = pl.pallas_call(
        _tbl_body,
        grid=(Bn,),
        in_specs=[
            pl.BlockSpec((1, rows_pad, Kp), lambda b: (b, 0, 0)),
            pl.BlockSpec((Kp, D), lambda b: (0, 0)),
            pl.BlockSpec((1, D), lambda b: (0, 0)),
            pl.BlockSpec((D, D), lambda b: (0, 0)),
        ],
        out_specs=pl.BlockSpec((1, rows_pad, D), lambda b: (b, 0, 0)),
        out_shape=jax.ShapeDtypeStruct((Bn, rows_pad, D), jnp.float32),
    )
    return call(x, Wp, bvec.reshape(1, D), conv_W)


def _head_body(x_ref, w1_ref, b1_ref, w2_ref, b2_ref, w3_ref, b3_ref, o_ref):
    x = jnp.tanh(x_ref[...])
    h = jnp.tanh(jnp.dot(x, w1_ref[...], preferred_element_type=jnp.float32)
                 + b1_ref[...])
    h = jnp.tanh(jnp.dot(h, w2_ref[...], preferred_element_type=jnp.float32)
                 + b2_ref[...])
    o_ref[...] = jnp.dot(h, w3_ref[...], preferred_element_type=jnp.float32) \
        + b3_ref[...]


def _head(cat, p1_W, p1_b, p2_W, p2_b, p3_W, p3_b):
    x = jnp.pad(cat, ((0, 8 - B), (0, 0)))
    w3 = jnp.pad(p3_W, ((0, 0), (0, D - 1)))
    b3 = jnp.pad(p3_b, ((0, D - 1),))
    call = pl.pallas_call(
        _head_body,
        out_shape=jax.ShapeDtypeStruct((8, D), jnp.float32),
    )
    out = call(x, p1_W, p1_b.reshape(1, -1), p2_W, p2_b.reshape(1, -1),
               w3, b3.reshape(1, D))
    return out[:B, 0]


def _gcn_body(table_ref, setcat_ref, esrc_ref, edst_ref, ids24_ref,
              tdiv_ref, adiv_ref, convb_ref, out_ref,
              slotmap, src2d, dst2d, mlist, rows, acc24, tw24, accrep,
              idx128, vals128, ones128, srcidx128, rid128, degs128, dinv128,
              initbf, lns, setbuf, idx32, ids24v, deg24, ns24, rid24, rep24,
              dinv24, logits, convbv, tdivv, adivv,
              node_src_sh, deg_sh, accsh):
    c = lax.axis_index("c")
    s = lax.axis_index("s")
    i16 = lax.broadcasted_iota(jnp.int32, (16,), 0)

    pltpu.sync_copy(convb_ref, convbv)
    pltpu.sync_copy(tdiv_ref, tdivv)
    pltpu.sync_copy(adiv_ref, adivv)
    for j in range(8):
        ones128[pl.ds(16 * j, 16)] = jnp.full((16,), 1.0, jnp.float32)
    idx32[pl.ds(0, 16)] = i16
    idx32[pl.ds(16, 16)] = i16 + 16

    def _fill_init(i, carry):
        initbf[pl.ds(i * 16, 16)] = jnp.full((16,), 1.0, jnp.float32)
        return carry
    lax.fori_loop(0, 40, _fill_init, 0)

    for stepi in range(2):
        b = c + 2 * stepi

        pltpu.sync_copy(initbf, deg_sh.at[pl.ds(s * 640, 640)])

        def _smfill(i, carry):
            slotmap[pl.ds(i * 16, 16)] = jnp.full((16,), -1, jnp.int32)
            return carry
        lax.fori_loop(0, NSARR // 16, _smfill, 0)

        for i in range(32):
            for j in range(8):
                acc24[i, pl.ds(16 * j, 16)] = jnp.zeros((16,), jnp.float32)

        @pl.when(s == 0)
        def _():
            pltpu.sync_copy(acc24, accsh)

        pltpu.sync_copy(ids24_ref.at[b], ids24v)
        ida = ids24v[pl.ds(0, 16)]
        idb = ids24v[pl.ds(16, 16)]
        plsc.store_scatter(slotmap, [ida], i16)
        plsc.store_scatter(slotmap, [idb], i16 + 16, mask=i16 < 8)

        pltpu.sync_copy(esrc_ref.at[b, s], src2d)
        pltpu.sync_copy(edst_ref.at[b, s], dst2d)
        plsc.subcore_barrier()

        lo = s * 640
        hi = lo + 640
        pltpu.sync_copy(setcat_ref.at[b], setbuf)

        def _lns_fill(i, carry):
            lns[pl.ds(i * 16, 16)] = jnp.full((16,), -1, jnp.int32)
            return carry
        lax.fori_loop(0, 40, _lns_fill, 0)

        def _p1(k, carry):
            base = k * 128
            for j in range(8):
                pos = base + 16 * j + i16
                iv = plsc.load_gather(setbuf, [pos])
                key = iv * 16 + i16
                ks, vs = plsc.sort_key_val(key, i16)
                nxt = jnp.take_along_axis(ks, jnp.minimum(i16 + 1, 15),
                                          axis=0)
                idx_s = lax.shift_right_logical(ks, 4)
                lastrun = (i16 == 15) | (lax.shift_right_logical(nxt, 4)
                                         != idx_s)
                val_s = base + 16 * j + vs
                inr = (idx_s >= lo) & (idx_s < hi) & lastrun
                plsc.store_scatter(lns, [idx_s - lo], val_s, mask=inr)
            return carry
        lax.fori_loop(0, SETP // 128, _p1, 0)
        pltpu.sync_copy(lns, node_src_sh.at[pl.ds(lo, 640)])

        def _p2(k, mcount):
            pltpu.sync_copy(ones128, deg_sh.at[dst2d.at[k]], add=True)
            kk = jnp.full((16,), k, jnp.int32)
            for j in range(8):
                d16 = plsc.load_gather(dst2d, [kk, i16 + 16 * j])
                s16 = plsc.load_gather(src2d, [kk, i16 + 16 * j])
                sl16 = plsc.load_gather(slotmap, [d16])
                m16 = sl16 >= 0
                packed = s16 | lax.shift_left(sl16, 14)
                plsc.store_compressed(mlist.at[pl.ds(mcount, 16)], packed,
                                      mask=m16)
                mcount = mcount + jnp.max(
                    plsc.all_reduce_population_count(m16))
            return mcount
        mcount = lax.fori_loop(0, NCHUNK, _p2, jnp.int32(0))
        plsc.subcore_barrier()

        padv = jnp.full((16,), 31 << 14, jnp.int32)
        for j in range(8):
            mlist[pl.ds(mcount + 16 * j, 16)] = padv
        nchunks = (mcount + 127) // 128

        def _p3(q, carry):
            for j in range(8):
                pk = plsc.load_gather(mlist, [q * 128 + 16 * j + i16])
                srcidx128[pl.ds(16 * j, 16)] = pk & 16383
            pltpu.sync_copy(node_src_sh.at[srcidx128], rid128)
            pltpu.sync_copy(deg_sh.at[srcidx128], degs128)
            for j in range(8):
                e16 = rid128[pl.ds(16 * j, 16)]
                e16 = jnp.where(e16 < 0, ZROW, e16)
                rid128[pl.ds(16 * j, 16)] = e16 + b * TROWS
            pltpu.sync_copy(table_ref.at[rid128], rows)
            for j in range(8):
                dinv128[pl.ds(16 * j, 16)] = _rsqrt16(degs128[pl.ds(16 * j, 16)])

            def _row(m, carry2):
                mm = jnp.full((16,), m, jnp.int32)
                dv = plsc.load_gather(dinv128, [mm])
                pk = plsc.load_gather(mlist, [q * 128 + mm])
                sl = lax.shift_right_logical(pk, 14)
                for j in range(8):
                    r16 = plsc.load_gather(rows, [mm, i16 + 16 * j])
                    plsc.addupdate_scatter(acc24, [sl, i16 + 16 * j], r16 * dv)
                return carry2
            lax.fori_loop(0, 128, _row, 0)
            return carry
        lax.fori_loop(0, nchunks, _p3, 0)

        pltpu.sync_copy(acc24, accsh.at[idx32], add=True)
        plsc.subcore_barrier()

        @pl.when(s == 0)
        def _():
            pltpu.sync_copy(deg_sh.at[ids24v], deg24)
            pltpu.sync_copy(node_src_sh.at[ids24v], ns24)
            for j in range(2):
                e16 = ns24[pl.ds(16 * j, 16)]
                e16 = jnp.where(e16 < 0, ZROW, e16)
                rid24[pl.ds(16 * j, 16)] = e16 + b * TROWS
            pltpu.sync_copy(table_ref.at[rid24], tw24)
            ida2 = ids24v[pl.ds(0, 16)]
            idb2 = ids24v[pl.ds(16, 16)]
            rpa = plsc.load_gather(slotmap, [ida2])
            rpb = plsc.load_gather(slotmap, [idb2])
            rpb = jnp.where(i16 < 8, rpb, 31)
            rep24[pl.ds(0, 16)] = rpa
            rep24[pl.ds(16, 16)] = rpb
            pltpu.sync_copy(accsh.at[rep24], accrep)
            for j in range(2):
                dinv24[pl.ds(16 * j, 16)] = _rsqrt16(deg24[pl.ds(16 * j, 16)])
            bb = jnp.full((16,), b, jnp.int32)
            tdsp = plsc.load_gather(tdivv, [bb])
            adsp = plsc.load_gather(adivv, [bb])
            zero = jnp.zeros((16,), jnp.float32)
            te = [zero] * 8
            ae1 = [zero] * 8
            ae0 = [zero] * 8
            for p in range(24):
                pp = jnp.full((16,), p, jnp.int32)
                dsp = plsc.load_gather(dinv24, [pp])
                isp = plsc.load_gather(ids24v, [pp])
                iz = isp == 0
                for j in range(8):
                    a16 = plsc.load_gather(accrep, [pp, i16 + 16 * j])
                    t16 = plsc.load_gather(tw24, [pp, i16 + 16 * j])
                    cb = convbv[pl.ds(16 * j, 16)]
                    row = cb + dsp * (a16 + t16 * dsp)
                    row = jnp.where(iz, jnp.float32(1e-05), row)
                    if p < 16:
                        te[j] = te[j] + row
                    elif p == 16:
                        ae0[j] = row
                    else:
                        ae1[j] = ae1[j] + row
            for j in range(8):
                logits[pl.ds(16 * j, 16)] = ae0[j]
                logits[pl.ds(128 + 16 * j, 16)] = ae1[j] / adsp
                logits[pl.ds(256 + 16 * j, 16)] = te[j] / tdsp
            pltpu.sync_copy(logits, out_ref.at[pl.ds(b * 384, 384)])
        plsc.subcore_barrier()


def _gcn(table, setcat, src_p, dst_p, ids24, tdiv, adiv, conv_b):
    mesh = plsc.VectorSubcoreMesh(core_axis_name="c", subcore_axis_name="s",
                                  num_cores=NC, num_subcores=NS)
    f = pl.kernel(
        _gcn_body,
        out_type=jax.ShapeDtypeStruct((B * 384,), jnp.float32),
        mesh=mesh,
        compiler_params=pltpu.CompilerParams(needs_layout_passes=False),
        scratch_types=[
            pltpu.VMEM((NSARR,), jnp.int32),
            pltpu.VMEM((NCHUNK, 128), jnp.int32),
            pltpu.VMEM((NCHUNK, 128), jnp.int32),
            pltpu.VMEM((MCAP,), jnp.int32),
            pltpu.VMEM((128, 128), jnp.float32),
            pltpu.VMEM((32, 128), jnp.float32),
            pltpu.VMEM((32, 128), jnp.float32),
            pltpu.VMEM((32, 128), jnp.float32),
            pltpu.VMEM((128,), jnp.int32),
            pltpu.VMEM((128,), jnp.int32),
            pltpu.VMEM((128,), jnp.float32),
            pltpu.VMEM((128,), jnp.int32),
            pltpu.VMEM((128,), jnp.int32),
            pltpu.VMEM((128,), jnp.float32),
            pltpu.VMEM((128,), jnp.float32),
            pltpu.VMEM((640,), jnp.float32),
            pltpu.VMEM((640,), jnp.int32),
            pltpu.VMEM((SETP,), jnp.int32),
            pltpu.VMEM((32,), jnp.int32),
            pltpu.VMEM((32,), jnp.int32),
            pltpu.VMEM((32,), jnp.float32),
            pltpu.VMEM((32,), jnp.int32),
            pltpu.VMEM((32,), jnp.int32),
            pltpu.VMEM((32,), jnp.int32),
            pltpu.VMEM((32,), jnp.float32),
            pltpu.VMEM((384,), jnp.float32),
            pltpu.VMEM((128,), jnp.float32),
            pltpu.VMEM((8,), jnp.float32),
            pltpu.VMEM((8,), jnp.float32),
            pltpu.VMEM_SHARED((NSARR,), jnp.int32),
            pltpu.VMEM_SHARED((NSARR,), jnp.float32),
            pltpu.VMEM_SHARED((32, 128), jnp.float32),
        ],
    )
    return f(table, setcat, src_p, dst_p, ids24, tdiv, adiv, conv_b)


def kernel(author_ids, topic_ids, auth_cnts, topic_cnts, paper_ids, id_maps,
           edge_indexs, paper_sets, author_sets, topic_sets,
           topic_embs, au_embs, paper_embs,
           topic_W, topic_b, au_W, au_b, paper_W, paper_b,
           conv_W, conv_b, p1_W, p1_b, p2_W, p2_b, p3_W, p3_b):
    t_t = _make_table(topic_embs, topic_W, topic_b, conv_W, 2000)
    t_a = _make_table(au_embs, au_W, au_b, conv_W, 3000)
    t_p = _make_table(paper_embs, paper_W, paper_b, conv_W, 5000)
    table = jnp.concatenate(
        [t_t, t_a, t_p[:, :4999], jnp.zeros((B, 1, D), jnp.float32)], axis=1)
    table = table.reshape(B * TROWS, D)

    setcat = jnp.concatenate([topic_sets, author_sets, paper_sets], axis=1)
    padn = SETP - setcat.shape[1]
    padidx = 10016 + (jnp.arange(padn, dtype=jnp.int32) % 128)
    setcat = jnp.concatenate(
        [setcat, jnp.broadcast_to(padidx, (B, padn))], axis=1)

    epad = NS * NCHUNK * 128 - E
    src_p = jnp.concatenate(
        [edge_indexs[:, 0, :], jnp.zeros((B, epad), jnp.int32)], axis=1)
    src_p = src_p.reshape(B, NS, NCHUNK, 128)
    dpad = 10016 + (jnp.arange(epad, dtype=jnp.int32) % 128)
    dst_p = jnp.concatenate(
        [edge_indexs[:, 1, :], jnp.broadcast_to(dpad, (B, epad))], axis=1)
    dst_p = dst_p.reshape(B, NS, NCHUNK, 128)

    ids24 = jnp.concatenate(
        [topic_ids, author_ids, jnp.zeros((B, 8), jnp.int32)], axis=1)
    tdiv = jnp.pad(np.float32(1e-05) + topic_cnts.astype(jnp.float32),
                   ((0, 8 - B),), constant_values=1.0)
    adiv = jnp.pad(np.float32(-1.00001) + auth_cnts.astype(jnp.float32),
                   ((0, 8 - B),), constant_values=1.0)

    precat = _gcn(table, setcat, src_p, dst_p, ids24, tdiv, adiv, conv_b)

    return _head(precat.reshape(B, 384), p1_W, p1_b, p2_W, p2_b, p3_W, p3_b)
